# baseline (speedup 1.0000x reference)
import functools

import jax
import jax.numpy as jnp
from jax import lax
from jax.experimental import pallas as pl
from jax.experimental.pallas import tpu as pltpu
from jax.experimental.pallas import tpu_sc as plsc

_N_NODES = 10000
_N_EDGES = 320000
_D = 128
_NC = 2
_NS = 16
_LANES = 16
_DH = _D // _NC
_EPT = _N_EDGES // _NS
_CHUNK = 128
_NCHUNK = 160
_EPAD = _NCHUNK * _CHUNK - _EPT
_NPAD = 10240
_RPT = _NPAD // _NS
_NG = 5

_mesh = plsc.VectorSubcoreMesh(
    core_axis_name="c", subcore_axis_name="s", num_cores=_NC, num_subcores=_NS
)


@functools.partial(
    pl.kernel,
    out_type=jax.ShapeDtypeStruct((_NC, _NPAD, _DH), jnp.float32),
    mesh=_mesh,
    scratch_types=[
        pltpu.VMEM((_NCHUNK, _CHUNK), jnp.int32),
        pltpu.VMEM((_NCHUNK, _CHUNK), jnp.int32),
        pltpu.VMEM((_CHUNK,), jnp.float32),
        pltpu.VMEM((_CHUNK,), jnp.float32),
        pltpu.VMEM((_CHUNK,), jnp.float32),
        pltpu.VMEM((_CHUNK,), jnp.float32),
        pltpu.VMEM((_CHUNK,), jnp.float32),
        pltpu.VMEM((_CHUNK, _DH), jnp.float32),
        pltpu.VMEM((_CHUNK, _DH), jnp.float32),
        pltpu.VMEM((_CHUNK, _DH), jnp.float32),
        pltpu.VMEM((_CHUNK, _DH), jnp.float32),
        pltpu.VMEM((_CHUNK, _DH), jnp.float32),
        pltpu.VMEM_SHARED((_NPAD, _DH), jnp.float32),
        pltpu.SemaphoreType.DMA,
        pltpu.SemaphoreType.DMA,
        pltpu.SemaphoreType.DMA,
        pltpu.SemaphoreType.DMA,
        pltpu.SemaphoreType.DMA,
        pltpu.SemaphoreType.DMA,
        pltpu.SemaphoreType.DMA,
        pltpu.SemaphoreType.DMA,
        pltpu.SemaphoreType.DMA,
        pltpu.SemaphoreType.DMA,
    ],
    compiler_params=pltpu.CompilerParams(
        needs_layout_passes=False, use_tc_tiling_on_sc=False
    ),
)
def _gsum_sc(x_cat, erow3, ecol3a, ecol3b, ev3, out, ridx, cidx,
             val0, val1, val2, val3, val4,
             rows0, rows1, rows2, rows3, rows4,
             acc, sg0, sg1, sg2, sg3, sg4, ss0, ss1, ss2, ss3, ss4):
    c = lax.axis_index("c")
    s = lax.axis_index("s")
    vals = (val0, val1, val2, val3, val4)
    rows = (rows0, rows1, rows2, rows3, rows4)
    semg = (sg0, sg1, sg2, sg3, sg4)
    sems = (ss0, ss1, ss2, ss3, ss4)

    pltpu.async_copy(erow3.at[s], ridx, sg0)

    @pl.when(c == 0)
    def _():
        pltpu.async_copy(ecol3a.at[s], cidx, sg1)

    @pl.when(c == 1)
    def _():
        pltpu.async_copy(ecol3b.at[s], cidx, sg1)


    zeros16 = jnp.zeros((_LANES,), jnp.float32)

    def _zero_row(i, carry):
        for j in range(_DH // _LANES):
            rows0[i, pl.ds(j * _LANES, _LANES)] = zeros16
        return carry

    lax.fori_loop(0, _CHUNK, _zero_row, 0)
    for k in range(_RPT // _CHUNK):
        pltpu.sync_copy(rows0, acc.at[pl.ds(s * _RPT + k * _CHUNK, _CHUNK)])

    pltpu.make_async_copy(erow3.at[s], ridx, sg0).wait()
    pltpu.make_async_copy(ecol3a.at[s], cidx, sg1).wait()
    plsc.subcore_barrier()

    def _issue_gather(ck, g):
        pltpu.async_copy(x_cat.at[cidx.at[ck]], rows[g], semg[g])
        pltpu.async_copy(ev3.at[s, ck], vals[g], semg[g])

    def _wait_gather(g):
        pltpu.make_async_copy(x_cat.at[cidx.at[0]], rows[g], semg[g]).wait()
        pltpu.make_async_copy(ev3.at[s, 0], vals[g], semg[g]).wait()

    def _issue_scatter(ck, g):
        pltpu.async_copy(rows[g], acc.at[ridx.at[ck]], sems[g], add=True)

    def _wait_scatter(g):
        pltpu.make_async_copy(rows[g], acc.at[ridx.at[0]], sems[g]).wait()

    def _scale(ck, g):
        rbuf = rows[g]
        vbuf = vals[g]

        def _grp(eb, carry):
            for k in range(8):
                e = eb * 8 + k
                vb = plsc.load_gather(vbuf, [jnp.full((_LANES,), e, jnp.int32)])
                for j in range(_DH // _LANES):
                    sl = pl.ds(j * _LANES, _LANES)
                    rbuf[e, sl] = rbuf[e, sl] * vb
            return carry

        lax.fori_loop(0, _CHUNK // 8, _grp, 0)

    _issue_gather(0, 0)
    _issue_gather(1, 1)
    _issue_gather(2, 2)

    def _quint(t, carry):
        for j in range(5):
            ck = t * 5 + j
            w = (j + 3) % _NG

            @pl.when(ck >= 2)
            def _():
                _wait_scatter(w)

            @pl.when(ck + 3 < _NCHUNK)
            def _():
                _issue_gather(ck + 3, w)

            _wait_gather(j)
            _scale(ck, j)
            _issue_scatter(ck, j)
        return carry

    lax.fori_loop(0, _NCHUNK // 5, _quint, 0)

    _wait_scatter(3)
    _wait_scatter(4)
    plsc.subcore_barrier()

    pltpu.sync_copy(acc.at[pl.ds(s * _RPT, _RPT)], out.at[c, pl.ds(s * _RPT, _RPT)])


def kernel(x, edge_index, edge_values):
    x_cat = jnp.concatenate([x[:, :_DH], x[:, _DH:]], axis=0)
    pad = ((0, 0), (0, _EPAD))
    erow3 = jnp.pad(
        edge_index[0].reshape(_NS, _EPT), pad, constant_values=_NPAD - 1
    ).reshape(_NS, _NCHUNK, _CHUNK)
    ecol3a = jnp.pad(
        edge_index[1].reshape(_NS, _EPT), pad, constant_values=0
    ).reshape(_NS, _NCHUNK, _CHUNK)
    ecol3b = ecol3a + _N_NODES
    ev3 = jnp.pad(
        edge_values.reshape(_NS, _EPT), pad, constant_values=0.0
    ).reshape(_NS, _NCHUNK, _CHUNK)
    out = _gsum_sc(x_cat, erow3, ecol3a, ecol3b, ev3)
    return jnp.concatenate([out[0, :_N_NODES], out[1, :_N_NODES]], axis=1)


# speedup vs baseline: 1.6514x; 1.6514x over previous
import functools

import jax
import jax.numpy as jnp
from jax import lax
from jax.experimental import pallas as pl
from jax.experimental.pallas import tpu as pltpu
from jax.experimental.pallas import tpu_sc as plsc

_N_NODES = 10000
_N_EDGES = 320000
_D = 128
_NC = 2
_NS = 16
_LANES = 16
_DH = _D // _NC
_EPT = _N_EDGES // _NS
_CHUNK = 80
_NCHUNK = 250
_EPAD = _NCHUNK * _CHUNK - _EPT
_NPAD = 10240
_RPT = _NPAD // _NS
_NG = 5

_mesh = plsc.VectorSubcoreMesh(
    core_axis_name="c", subcore_axis_name="s", num_cores=_NC, num_subcores=_NS
)


@functools.partial(
    pl.kernel,
    out_type=jax.ShapeDtypeStruct((_NC, _NPAD, _DH), jnp.float32),
    mesh=_mesh,
    scratch_types=[
        pltpu.VMEM((_NCHUNK, _CHUNK), jnp.int32),
        pltpu.VMEM((_NCHUNK, _CHUNK), jnp.int32),
        pltpu.VMEM((_NCHUNK, _CHUNK), jnp.float32),
        pltpu.VMEM((_CHUNK, _DH), jnp.float32),
        pltpu.VMEM((_CHUNK, _DH), jnp.float32),
        pltpu.VMEM((_CHUNK, _DH), jnp.float32),
        pltpu.VMEM((_CHUNK, _DH), jnp.float32),
        pltpu.VMEM((_CHUNK, _DH), jnp.float32),
        pltpu.VMEM_SHARED((_NPAD, _DH), jnp.float32),
        pltpu.SemaphoreType.DMA,
        pltpu.SemaphoreType.DMA,
        pltpu.SemaphoreType.DMA,
        pltpu.SemaphoreType.DMA,
        pltpu.SemaphoreType.DMA,
        pltpu.SemaphoreType.DMA,
        pltpu.SemaphoreType.DMA,
        pltpu.SemaphoreType.DMA,
        pltpu.SemaphoreType.DMA,
        pltpu.SemaphoreType.DMA,
    ],
    compiler_params=pltpu.CompilerParams(
        needs_layout_passes=False, use_tc_tiling_on_sc=False
    ),
)
def _gsum_sc(x_cat, erow3, ecol3a, ecol3b, ev3, out, ridx, cidx, val,
             rows0, rows1, rows2, rows3, rows4,
             acc, sg0, sg1, sg2, sg3, sg4, ss0, ss1, ss2, ss3, ss4):
    c = lax.axis_index("c")
    s = lax.axis_index("s")
    rows = (rows0, rows1, rows2, rows3, rows4)
    semg = (sg0, sg1, sg2, sg3, sg4)
    sems = (ss0, ss1, ss2, ss3, ss4)

    pltpu.async_copy(erow3.at[s], ridx, sg0)

    @pl.when(c == 0)
    def _():
        pltpu.async_copy(ecol3a.at[s], cidx, sg1)

    @pl.when(c == 1)
    def _():
        pltpu.async_copy(ecol3b.at[s], cidx, sg1)

    pltpu.async_copy(ev3.at[s], val, sg2)

    zeros16 = jnp.zeros((_LANES,), jnp.float32)

    def _zero_row(i, carry):
        for j in range(_DH // _LANES):
            rows0[i, pl.ds(j * _LANES, _LANES)] = zeros16
        return carry

    lax.fori_loop(0, _CHUNK, _zero_row, 0)
    for k in range(_RPT // _CHUNK):
        pltpu.sync_copy(rows0, acc.at[pl.ds(s * _RPT + k * _CHUNK, _CHUNK)])

    pltpu.make_async_copy(erow3.at[s], ridx, sg0).wait()
    pltpu.make_async_copy(ecol3a.at[s], cidx, sg1).wait()
    pltpu.make_async_copy(ev3.at[s], val, sg2).wait()
    plsc.subcore_barrier()

    def _issue_gather(ck, g):
        pltpu.async_copy(x_cat.at[cidx.at[ck]], rows[g], semg[g])

    def _wait_gather(g):
        pltpu.make_async_copy(x_cat.at[cidx.at[0]], rows[g], semg[g]).wait()

    def _issue_scatter(ck, g):
        pltpu.async_copy(rows[g], acc.at[ridx.at[ck]], sems[g], add=True)

    def _wait_scatter(g):
        pltpu.make_async_copy(rows[g], acc.at[ridx.at[0]], sems[g]).wait()

    def _scale(ck, g):
        rbuf = rows[g]
        i0 = jnp.full((_LANES,), ck, jnp.int32)

        def _grp(eb, carry):
            for k in range(8):
                e = eb * 8 + k
                vb = plsc.load_gather(val, [i0, jnp.full((_LANES,), e, jnp.int32)])
                for j in range(_DH // _LANES):
                    sl = pl.ds(j * _LANES, _LANES)
                    rbuf[e, sl] = rbuf[e, sl] * vb
            return carry

        lax.fori_loop(0, _CHUNK // 8, _grp, 0)

    _issue_gather(0, 0)
    _issue_gather(1, 1)
    _issue_gather(2, 2)

    def _quint(t, carry):
        for j in range(5):
            ck = t * 5 + j
            w = (j + 3) % _NG

            @pl.when(ck >= 2)
            def _():
                _wait_scatter(w)

            @pl.when(ck + 3 < _NCHUNK)
            def _():
                _issue_gather(ck + 3, w)

            _wait_gather(j)
            _scale(ck, j)
            _issue_scatter(ck, j)
        return carry

    lax.fori_loop(0, _NCHUNK // 5, _quint, 0)

    _wait_scatter(3)
    _wait_scatter(4)
    plsc.subcore_barrier()

    pltpu.sync_copy(acc.at[pl.ds(s * _RPT, _RPT)], out.at[c, pl.ds(s * _RPT, _RPT)])


def kernel(x, edge_index, edge_values):
    x_cat = jnp.concatenate([x[:, :_DH], x[:, _DH:]], axis=0)
    pad = ((0, 0), (0, _EPAD))
    erow3 = jnp.pad(
        edge_index[0].reshape(_NS, _EPT), pad, constant_values=_NPAD - 1
    ).reshape(_NS, _NCHUNK, _CHUNK)
    ecol3a = jnp.pad(
        edge_index[1].reshape(_NS, _EPT), pad, constant_values=0
    ).reshape(_NS, _NCHUNK, _CHUNK)
    ecol3b = ecol3a + _N_NODES
    ev3 = jnp.pad(
        edge_values.reshape(_NS, _EPT), pad, constant_values=0.0
    ).reshape(_NS, _NCHUNK, _CHUNK)
    out = _gsum_sc(x_cat, erow3, ecol3a, ecol3b, ev3)
    return jnp.concatenate([out[0, :_N_NODES], out[1, :_N_NODES]], axis=1)
